# baseline (speedup 1.0000x reference)
import functools

import jax
import jax.numpy as jnp
import numpy as np
from jax import lax
from jax.experimental import pallas as pl
from jax.experimental.pallas import tpu as pltpu
from jax.experimental.pallas import tpu_sc as plsc

N = 10000
NPAD = 10240
IN_DIM = 256
FEAT = 256
LAT = 64
E = 160000
EPAD = 163840

NC, NS, L = 2, 16, 16
CHUNK = 128
CHUNKS = EPAD // NS // CHUNK
RPT = NPAD // NS
EPW = EPAD // (NC * NS)


def _mesh():
    return plsc.VectorSubcoreMesh(
        core_axis_name="c", subcore_axis_name="s",
        num_cores=NC, num_subcores=NS)


def _deg_body(dst_hbm, hist_hbm, dst_v, hist_v):
    c = lax.axis_index("c")
    s = lax.axis_index("s")
    w = s * NC + c
    pltpu.sync_copy(dst_hbm.at[w], dst_v)
    zeros = jnp.zeros((L,), jnp.float32)
    ones = jnp.ones((L,), jnp.float32)

    @pl.loop(0, NPAD // L)
    def _zero(i):
        hist_v[pl.ds(i * L, L)] = zeros

    @pl.loop(0, EPW // L)
    def _count(i):
        idx = dst_v[pl.ds(i * L, L)]
        plsc.addupdate_scatter(hist_v, [idx], ones)

    pltpu.sync_copy(hist_v, hist_hbm.at[w])


@functools.cache
def _deg_kernel():
    return functools.partial(
        pl.kernel,
        out_type=jax.ShapeDtypeStruct((NC * NS, NPAD), jnp.float32),
        mesh=_mesh(),
        scratch_types=[
            pltpu.VMEM((EPW,), jnp.int32),
            pltpu.VMEM((NPAD,), jnp.float32),
        ],
        compiler_params=pltpu.CompilerParams(needs_layout_passes=False),
    )(_deg_body)


def _make_scatter(edge_split):
    DH = 128
    chunks = CHUNKS // 2 if edge_split else CHUNKS

    def body(table_hbm, src_hbm, dst_hbm, out_hbm,
             src_a, src_b, dst_v, rows_a, rows_b,
             sem_ga, sem_gb, sem_ia, sem_ib, sem_sa, sem_sb, acc):
        c = lax.axis_index("c")
        s = lax.axis_index("s")
        w = c * NS + s if edge_split else s
        pltpu.sync_copy(dst_hbm.at[w], dst_v)

        zeros = jnp.zeros((L,), jnp.float32)

        @pl.loop(0, CHUNK)
        def _zr(r):
            @pl.loop(0, DH // L)
            def _zc(k):
                rows_a[r, pl.ds(k * L, L)] = zeros

        for k in range(RPT // CHUNK):
            pltpu.sync_copy(rows_a, acc.at[pl.ds(s * RPT + k * CHUNK, CHUNK)])

        table = table_hbm if edge_split else table_hbm.at[c]
        last = chunks - 1

        def gather(sbuf, rbuf, sem):
            pltpu.async_copy(table.at[sbuf], rbuf, sem)

        def gwait(rbuf, sem):
            pltpu.make_async_copy(table.at[src_a], rbuf, sem).wait()

        def iload(jj, sbuf, sem):
            pltpu.async_copy(src_hbm.at[w, jj], sbuf, sem)

        def iwait(sbuf, sem):
            pltpu.make_async_copy(src_hbm.at[w, 0], sbuf, sem).wait()

        def scat(rbuf, jj, sem):
            pltpu.async_copy(rbuf, acc.at[dst_v.at[jj]], sem, add=True)

        def swait(rbuf, sem):
            pltpu.make_async_copy(table.at[src_a], rbuf, sem).wait()

        pltpu.sync_copy(src_hbm.at[w, 0], src_a)
        gather(src_a, rows_a, sem_ga)
        iload(1, src_b, sem_ib)
        iwait(src_b, sem_ib)
        plsc.subcore_barrier()

        gwait(rows_a, sem_ga)
        scat(rows_a, 0, sem_sa)
        gather(src_b, rows_b, sem_gb)
        iload(2, src_a, sem_ia)
        gwait(rows_b, sem_gb)
        scat(rows_b, 1, sem_sb)
        swait(rows_a, sem_sa)
        iwait(src_a, sem_ia)
        gather(src_a, rows_a, sem_ga)
        iload(3, src_b, sem_ib)
        iwait(src_b, sem_ib)

        @pl.loop(2, chunks, step=2)
        def _edge(j):
            gwait(rows_a, sem_ga)
            scat(rows_a, j, sem_sa)
            swait(rows_b, sem_sb)
            gather(src_b, rows_b, sem_gb)
            iload(jnp.minimum(j + 2, last), src_a, sem_ia)
            gwait(rows_b, sem_gb)
            scat(rows_b, j + 1, sem_sb)
            swait(rows_a, sem_sa)
            iwait(src_a, sem_ia)
            gather(src_a, rows_a, sem_ga)
            iload(jnp.minimum(j + 3, last), src_b, sem_ib)
            iwait(src_b, sem_ib)

        swait(rows_b, sem_sb)
        gwait(rows_a, sem_ga)
        plsc.subcore_barrier()
        pltpu.sync_copy(acc.at[pl.ds(s * RPT, RPT)],
                        out_hbm.at[c, pl.ds(s * RPT, RPT)])

    tshape = (NPAD, DH) if edge_split else (NC, NPAD, DH)
    nw = NC * NS if edge_split else NS
    return functools.partial(
        pl.kernel,
        out_type=jax.ShapeDtypeStruct((NC, NPAD, DH), jnp.float32),
        mesh=_mesh(),
        scratch_types=[
            pltpu.VMEM((CHUNK,), jnp.int32),
            pltpu.VMEM((CHUNK,), jnp.int32),
            pltpu.VMEM((chunks, CHUNK), jnp.int32),
            pltpu.VMEM((CHUNK, DH), jnp.float32),
            pltpu.VMEM((CHUNK, DH), jnp.float32),
            pltpu.SemaphoreType.DMA,
            pltpu.SemaphoreType.DMA,
            pltpu.SemaphoreType.DMA,
            pltpu.SemaphoreType.DMA,
            pltpu.SemaphoreType.DMA,
            pltpu.SemaphoreType.DMA,
            pltpu.VMEM_SHARED((NPAD, DH), jnp.float32),
        ],
        compiler_params=pltpu.CompilerParams(needs_layout_passes=False),
    )(body)


_scatter_kernel = functools.cache(_make_scatter)


_BM = 512


def _prescale_body(hist_ref, x_ref, dinv_ref, ycat_ref):
    i = pl.program_id(0)
    deg = jnp.sum(hist_ref[...], axis=0) + 1.0
    dinv = lax.rsqrt(jnp.maximum(deg, 1.0))
    y = x_ref[...] * dinv[:, None]
    row = i * _BM + lax.broadcasted_iota(jnp.int32, (_BM, 1), 0)
    y = jnp.where(row < N, y, 0.0)
    dinv_ref[...] = dinv
    ycat_ref[0] = y[:, : IN_DIM // 2]
    ycat_ref[1] = y[:, IN_DIM // 2:]


def _prescale(hist, x):
    return pl.pallas_call(
        _prescale_body,
        grid=(NPAD // _BM,),
        in_specs=[pl.BlockSpec((NC * NS, _BM), lambda i: (0, i)),
                  pl.BlockSpec((_BM, IN_DIM), lambda i: (i, 0))],
        out_specs=[pl.BlockSpec((_BM,), lambda i: (i,)),
                   pl.BlockSpec((NC, _BM, IN_DIM // 2), lambda i: (0, i, 0))],
        out_shape=[jax.ShapeDtypeStruct((NPAD,), jnp.float32),
                   jax.ShapeDtypeStruct((NC, NPAD, IN_DIM // 2), jnp.float32)],
    )(hist, x)


def _enc_body(sx_ref, yx_ref, dinv_ref, b1_ref, w1_ref, wcat_ref, y2_ref):
    i = pl.program_id(0)
    sx = jnp.concatenate([sx_ref[0], sx_ref[1]], axis=1)
    yx = jnp.concatenate([yx_ref[0], yx_ref[1]], axis=1)
    dinv = dinv_ref[...]
    pre = dinv[:, None] * (sx + yx)
    h = jnp.maximum(
        jnp.dot(pre, w1_ref[...], preferred_element_type=jnp.float32)
        + b1_ref[...][None, :], 0.0)
    row = i * _BM + lax.broadcasted_iota(jnp.int32, (_BM, 1), 0)
    h = jnp.where(row < N, h, 0.0)
    c = jnp.dot(h, wcat_ref[...], preferred_element_type=jnp.float32)
    y2_ref[...] = c * dinv[:, None]


def _enc(sxcat, yxcat, dinv, b1, w1, wcat):
    return pl.pallas_call(
        _enc_body,
        grid=(NPAD // _BM,),
        in_specs=[pl.BlockSpec((NC, _BM, IN_DIM // 2), lambda i: (0, i, 0)),
                  pl.BlockSpec((NC, _BM, IN_DIM // 2), lambda i: (0, i, 0)),
                  pl.BlockSpec((_BM,), lambda i: (i,)),
                  pl.BlockSpec((FEAT,), lambda i: (0,)),
                  pl.BlockSpec((IN_DIM, FEAT), lambda i: (0, 0)),
                  pl.BlockSpec((FEAT, 2 * LAT), lambda i: (0, 0))],
        out_specs=pl.BlockSpec((_BM, 2 * LAT), lambda i: (i, 0)),
        out_shape=jax.ShapeDtypeStruct((NPAD, 2 * LAT), jnp.float32),
    )(sxcat, yxcat, dinv, b1, w1, wcat)


def _zcomp_body(s2_ref, y2_ref, dinv_ref, bcat_ref, gn_ref, z_ref):
    s = s2_ref[0] + s2_ref[1]
    y2 = y2_ref[...]
    o = dinv_ref[...][:, None] * (s + y2) + bcat_ref[...][None, :]
    xu = o[:, :LAT]
    xs = o[:, LAT:]
    z = gn_ref[...] * jnp.exp(xs) + xu
    z_ref[...] = (z * 0.7071067811865476).astype(jnp.bfloat16)


def _zcomp(s2cat, y2cat, dinv, bcat, gn):
    return pl.pallas_call(
        _zcomp_body,
        grid=(NPAD // _BM,),
        in_specs=[pl.BlockSpec((NC, _BM, 2 * LAT), lambda i: (0, i, 0)),
                  pl.BlockSpec((_BM, 2 * LAT), lambda i: (i, 0)),
                  pl.BlockSpec((_BM,), lambda i: (i,)),
                  pl.BlockSpec((2 * LAT,), lambda i: (0,)),
                  pl.BlockSpec((_BM, LAT), lambda i: (i, 0))],
        out_specs=pl.BlockSpec((_BM, LAT), lambda i: (i, 0)),
        out_shape=jax.ShapeDtypeStruct((NPAD, LAT), jnp.bfloat16),
    )(s2cat, y2cat, dinv, bcat, gn)


_BD = 2048
_BDN = 2048


def _decode_body(zr_ref, zc_ref, o_ref):
    p = lax.dot_general(zr_ref[...], zc_ref[...],
                        (((1,), (1,)), ((), ())),
                        preferred_element_type=jnp.float32)
    o_ref[...] = 0.5 * jnp.tanh(p) + 0.5


def _decode(z):
    return pl.pallas_call(
        _decode_body,
        grid=(NPAD // _BD, NPAD // _BDN),
        in_specs=[pl.BlockSpec((_BD, LAT), lambda i, j: (i, 0)),
                  pl.BlockSpec((_BDN, LAT), lambda i, j: (j, 0))],
        out_specs=pl.BlockSpec((_BD, _BDN), lambda i, j: (i, j)),
        out_shape=jax.ShapeDtypeStruct((N, N), jnp.float32),
    )(z, z)


def kernel(x, edge_index, W1, b1, Wmu, bmu, Wsig, bsig, gnoise):
    ei = edge_index.astype(jnp.int32)
    pad = jnp.asarray(N + np.arange(EPAD - E, dtype=np.int32) % (NPAD - N))
    src = jnp.concatenate([ei[0], pad])
    dst = jnp.concatenate([ei[1], pad])
    src_sc = src.reshape(NS, CHUNKS, CHUNK)
    dst_sc = dst.reshape(NS, CHUNKS, CHUNK)
    src_es = src.reshape(NC * NS, CHUNKS // 2, CHUNK)
    dst_es = dst.reshape(NC * NS, CHUNKS // 2, CHUNK)
    dst_deg = dst.reshape(NC * NS, EPW)

    wcat = jnp.concatenate([Wmu, Wsig], axis=1)
    bcat = jnp.concatenate([bmu, bsig])

    hist = _deg_kernel()(dst_deg)
    dinv, yxcat = _prescale(hist, x)
    sxcat = _scatter_kernel(False)(yxcat, src_sc, dst_sc)
    y2 = _enc(sxcat, yxcat, dinv, b1, W1, wcat)
    s2cat = _scatter_kernel(True)(y2, src_es, dst_es)
    z = _zcomp(s2cat, y2, dinv, bcat, gnoise)
    return _decode(z)


# speedup vs baseline: 1.0006x; 1.0006x over previous
import functools

import jax
import jax.numpy as jnp
import numpy as np
from jax import lax
from jax.experimental import pallas as pl
from jax.experimental.pallas import tpu as pltpu
from jax.experimental.pallas import tpu_sc as plsc

N = 10000
NPAD = 10240
IN_DIM = 256
FEAT = 256
LAT = 64
E = 160000
EPAD = 163840

NC, NS, L = 2, 16, 16
CHUNK = 128
CHUNKS = EPAD // NS // CHUNK
RPT = NPAD // NS
EPW = EPAD // (NC * NS)


def _mesh():
    return plsc.VectorSubcoreMesh(
        core_axis_name="c", subcore_axis_name="s",
        num_cores=NC, num_subcores=NS)


def _deg_body(dst_hbm, hist_hbm, dst_v, hist_v):
    c = lax.axis_index("c")
    s = lax.axis_index("s")
    w = s * NC + c
    pltpu.sync_copy(dst_hbm.at[w], dst_v)
    zeros = jnp.zeros((L,), jnp.float32)
    ones = jnp.ones((L,), jnp.float32)

    @pl.loop(0, NPAD // L)
    def _zero(i):
        hist_v[pl.ds(i * L, L)] = zeros

    @pl.loop(0, EPW // L)
    def _count(i):
        idx = dst_v[pl.ds(i * L, L)]
        plsc.addupdate_scatter(hist_v, [idx], ones)

    pltpu.sync_copy(hist_v, hist_hbm.at[w])


@functools.cache
def _deg_kernel():
    return functools.partial(
        pl.kernel,
        out_type=jax.ShapeDtypeStruct((NC * NS, NPAD), jnp.float32),
        mesh=_mesh(),
        scratch_types=[
            pltpu.VMEM((EPW,), jnp.int32),
            pltpu.VMEM((NPAD,), jnp.float32),
        ],
        compiler_params=pltpu.CompilerParams(needs_layout_passes=False),
    )(_deg_body)


def _make_scatter(edge_split):
    DH = 128
    chunks = CHUNKS // 2 if edge_split else CHUNKS

    def body(table_hbm, src_hbm, dst_hbm, out_hbm,
             src_a, src_b, dst_v, rows_a, rows_b,
             sem_ga, sem_gb, sem_ia, sem_ib, sem_sa, sem_sb, acc):
        c = lax.axis_index("c")
        s = lax.axis_index("s")
        w = c * NS + s if edge_split else s
        pltpu.sync_copy(dst_hbm.at[w], dst_v)

        zeros = jnp.zeros((L,), jnp.float32)

        @pl.loop(0, CHUNK)
        def _zr(r):
            @pl.loop(0, DH // L)
            def _zc(k):
                rows_a[r, pl.ds(k * L, L)] = zeros

        for k in range(RPT // CHUNK):
            pltpu.sync_copy(rows_a, acc.at[pl.ds(s * RPT + k * CHUNK, CHUNK)])

        table = table_hbm if edge_split else table_hbm.at[c]
        last = chunks - 1

        def gather(sbuf, rbuf, sem):
            pltpu.async_copy(table.at[sbuf], rbuf, sem)

        def gwait(rbuf, sem):
            pltpu.make_async_copy(table.at[src_a], rbuf, sem).wait()

        def iload(jj, sbuf, sem):
            pltpu.async_copy(src_hbm.at[w, jj], sbuf, sem)

        def iwait(sbuf, sem):
            pltpu.make_async_copy(src_hbm.at[w, 0], sbuf, sem).wait()

        def scat(rbuf, jj, sem):
            pltpu.async_copy(rbuf, acc.at[dst_v.at[jj]], sem, add=True)

        def swait(rbuf, sem):
            pltpu.make_async_copy(table.at[src_a], rbuf, sem).wait()

        pltpu.sync_copy(src_hbm.at[w, 0], src_a)
        gather(src_a, rows_a, sem_ga)
        iload(1, src_b, sem_ib)
        iwait(src_b, sem_ib)
        plsc.subcore_barrier()

        gwait(rows_a, sem_ga)
        scat(rows_a, 0, sem_sa)
        gather(src_b, rows_b, sem_gb)
        iload(2, src_a, sem_ia)
        gwait(rows_b, sem_gb)
        scat(rows_b, 1, sem_sb)
        swait(rows_a, sem_sa)
        iwait(src_a, sem_ia)
        gather(src_a, rows_a, sem_ga)
        iload(3, src_b, sem_ib)
        iwait(src_b, sem_ib)

        @pl.loop(2, chunks, step=2)
        def _edge(j):
            gwait(rows_a, sem_ga)
            scat(rows_a, j, sem_sa)
            swait(rows_b, sem_sb)
            gather(src_b, rows_b, sem_gb)
            iload(jnp.minimum(j + 2, last), src_a, sem_ia)
            gwait(rows_b, sem_gb)
            scat(rows_b, j + 1, sem_sb)
            swait(rows_a, sem_sa)
            iwait(src_a, sem_ia)
            gather(src_a, rows_a, sem_ga)
            iload(jnp.minimum(j + 3, last), src_b, sem_ib)
            iwait(src_b, sem_ib)

        swait(rows_b, sem_sb)
        gwait(rows_a, sem_ga)
        plsc.subcore_barrier()
        pltpu.sync_copy(acc.at[pl.ds(s * RPT, RPT)],
                        out_hbm.at[c, pl.ds(s * RPT, RPT)])

    return functools.partial(
        pl.kernel,
        out_type=jax.ShapeDtypeStruct((NC, NPAD, DH), jnp.float32),
        mesh=_mesh(),
        scratch_types=[
            pltpu.VMEM((CHUNK,), jnp.int32),
            pltpu.VMEM((CHUNK,), jnp.int32),
            pltpu.VMEM((chunks, CHUNK), jnp.int32),
            pltpu.VMEM((CHUNK, DH), jnp.float32),
            pltpu.VMEM((CHUNK, DH), jnp.float32),
            pltpu.SemaphoreType.DMA,
            pltpu.SemaphoreType.DMA,
            pltpu.SemaphoreType.DMA,
            pltpu.SemaphoreType.DMA,
            pltpu.SemaphoreType.DMA,
            pltpu.SemaphoreType.DMA,
            pltpu.VMEM_SHARED((NPAD, DH), jnp.float32),
        ],
        compiler_params=pltpu.CompilerParams(needs_layout_passes=False),
    )(body)


_scatter_kernel = functools.cache(_make_scatter)


_BM = 512


def _prescale_body(hist_ref, x_ref, dinv_ref, ycat_ref):
    i = pl.program_id(0)
    deg = jnp.sum(hist_ref[...], axis=0) + 1.0
    dinv = lax.rsqrt(jnp.maximum(deg, 1.0))
    y = x_ref[...] * dinv[:, None]
    row = i * _BM + lax.broadcasted_iota(jnp.int32, (_BM, 1), 0)
    y = jnp.where(row < N, y, 0.0)
    dinv_ref[...] = dinv
    ycat_ref[0] = y[:, : IN_DIM // 2]
    ycat_ref[1] = y[:, IN_DIM // 2:]


def _prescale(hist, x):
    return pl.pallas_call(
        _prescale_body,
        grid=(NPAD // _BM,),
        in_specs=[pl.BlockSpec((NC * NS, _BM), lambda i: (0, i)),
                  pl.BlockSpec((_BM, IN_DIM), lambda i: (i, 0))],
        out_specs=[pl.BlockSpec((_BM,), lambda i: (i,)),
                   pl.BlockSpec((NC, _BM, IN_DIM // 2), lambda i: (0, i, 0))],
        out_shape=[jax.ShapeDtypeStruct((NPAD,), jnp.float32),
                   jax.ShapeDtypeStruct((NC, NPAD, IN_DIM // 2), jnp.float32)],
    )(hist, x)


def _enc_body(sx_ref, yx_ref, dinv_ref, b1_ref, w1_ref, wcat_ref, y2_ref):
    i = pl.program_id(0)
    sx = jnp.concatenate([sx_ref[0], sx_ref[1]], axis=1)
    yx = jnp.concatenate([yx_ref[0], yx_ref[1]], axis=1)
    dinv = dinv_ref[...]
    pre = dinv[:, None] * (sx + yx)
    h = jnp.maximum(
        jnp.dot(pre, w1_ref[...], preferred_element_type=jnp.float32)
        + b1_ref[...][None, :], 0.0)
    row = i * _BM + lax.broadcasted_iota(jnp.int32, (_BM, 1), 0)
    h = jnp.where(row < N, h, 0.0)
    c = jnp.dot(h, wcat_ref[...], preferred_element_type=jnp.float32)
    y2_ref[...] = c * dinv[:, None]


def _enc(sxcat, yxcat, dinv, b1, w1, wcat):
    return pl.pallas_call(
        _enc_body,
        grid=(NPAD // _BM,),
        in_specs=[pl.BlockSpec((NC, _BM, IN_DIM // 2), lambda i: (0, i, 0)),
                  pl.BlockSpec((NC, _BM, IN_DIM // 2), lambda i: (0, i, 0)),
                  pl.BlockSpec((_BM,), lambda i: (i,)),
                  pl.BlockSpec((FEAT,), lambda i: (0,)),
                  pl.BlockSpec((IN_DIM, FEAT), lambda i: (0, 0)),
                  pl.BlockSpec((FEAT, 2 * LAT), lambda i: (0, 0))],
        out_specs=pl.BlockSpec((_BM, 2 * LAT), lambda i: (i, 0)),
        out_shape=jax.ShapeDtypeStruct((NPAD, 2 * LAT), jnp.float32),
    )(sxcat, yxcat, dinv, b1, w1, wcat)


def _zcomp_body(s2_ref, y2_ref, dinv_ref, bcat_ref, gn_ref, z_ref):
    s = s2_ref[0] + s2_ref[1]
    y2 = y2_ref[...]
    o = dinv_ref[...][:, None] * (s + y2) + bcat_ref[...][None, :]
    xu = o[:, :LAT]
    xs = o[:, LAT:]
    z = gn_ref[...] * jnp.exp(xs) + xu
    z_ref[...] = (z * 0.7071067811865476).astype(jnp.bfloat16)


def _zcomp(s2cat, y2cat, dinv, bcat, gn):
    return pl.pallas_call(
        _zcomp_body,
        grid=(NPAD // _BM,),
        in_specs=[pl.BlockSpec((NC, _BM, 2 * LAT), lambda i: (0, i, 0)),
                  pl.BlockSpec((_BM, 2 * LAT), lambda i: (i, 0)),
                  pl.BlockSpec((_BM,), lambda i: (i,)),
                  pl.BlockSpec((2 * LAT,), lambda i: (0,)),
                  pl.BlockSpec((_BM, LAT), lambda i: (i, 0))],
        out_specs=pl.BlockSpec((_BM, LAT), lambda i: (i, 0)),
        out_shape=jax.ShapeDtypeStruct((NPAD, LAT), jnp.bfloat16),
    )(s2cat, y2cat, dinv, bcat, gn)


_BD = 2048
_BDN = 2048


def _decode_body(zr_ref, zc_ref, o_ref):
    p = lax.dot_general(zr_ref[...], zc_ref[...],
                        (((1,), (1,)), ((), ())),
                        preferred_element_type=jnp.float32)
    o_ref[...] = 0.5 * jnp.tanh(p) + 0.5


def _decode(z):
    return pl.pallas_call(
        _decode_body,
        grid=(NPAD // _BD, NPAD // _BDN),
        in_specs=[pl.BlockSpec((_BD, LAT), lambda i, j: (i, 0)),
                  pl.BlockSpec((_BDN, LAT), lambda i, j: (j, 0))],
        out_specs=pl.BlockSpec((_BD, _BDN), lambda i, j: (i, j)),
        out_shape=jax.ShapeDtypeStruct((N, N), jnp.float32),
    )(z, z)


def kernel(x, edge_index, W1, b1, Wmu, bmu, Wsig, bsig, gnoise):
    ei = edge_index.astype(jnp.int32)
    pad = jnp.asarray(N + np.arange(EPAD - E, dtype=np.int32) % (NPAD - N))
    src = jnp.concatenate([ei[0], pad])
    dst = jnp.concatenate([ei[1], pad])
    src_sc = src.reshape(NS, CHUNKS, CHUNK)
    dst_sc = dst.reshape(NS, CHUNKS, CHUNK)
    src_es = src.reshape(NC * NS, CHUNKS // 2, CHUNK)
    dst_es = dst.reshape(NC * NS, CHUNKS // 2, CHUNK)
    dst_deg = dst.reshape(NC * NS, EPW)

    wcat = jnp.concatenate([Wmu, Wsig], axis=1)
    bcat = jnp.concatenate([bmu, bsig])

    hist = _deg_kernel()(dst_deg)
    dinv, yxcat = _prescale(hist, x)
    sxcat = _scatter_kernel(False)(yxcat, src_sc, dst_sc)
    y2 = _enc(sxcat, yxcat, dinv, b1, W1, wcat)
    s2cat = _scatter_kernel(True)(y2, src_es, dst_es)
    z = _zcomp(s2cat, y2, dinv, bcat, gnoise)
    return _decode(z)


# speedup vs baseline: 1.0008x; 1.0001x over previous
import functools

import jax
import jax.numpy as jnp
import numpy as np
from jax import lax
from jax.experimental import pallas as pl
from jax.experimental.pallas import tpu as pltpu
from jax.experimental.pallas import tpu_sc as plsc

N = 10000
NPAD = 10240
IN_DIM = 256
FEAT = 256
LAT = 64
E = 160000
EPAD = 163840

NC, NS, L = 2, 16, 16
CHUNK = 128
CHUNKS = EPAD // NS // CHUNK
RPT = NPAD // NS
EPW = EPAD // (NC * NS)


def _mesh():
    return plsc.VectorSubcoreMesh(
        core_axis_name="c", subcore_axis_name="s",
        num_cores=NC, num_subcores=NS)


def _deg_body(dst_hbm, hist_hbm, dst_v, hist_v):
    c = lax.axis_index("c")
    s = lax.axis_index("s")
    w = s * NC + c
    pltpu.sync_copy(dst_hbm.at[w], dst_v)
    zeros = jnp.zeros((L,), jnp.float32)
    ones = jnp.ones((L,), jnp.float32)

    @pl.loop(0, NPAD // L)
    def _zero(i):
        hist_v[pl.ds(i * L, L)] = zeros

    @pl.loop(0, EPW // L)
    def _count(i):
        idx = dst_v[pl.ds(i * L, L)]
        plsc.addupdate_scatter(hist_v, [idx], ones)

    pltpu.sync_copy(hist_v, hist_hbm.at[w])


@functools.cache
def _deg_kernel():
    return functools.partial(
        pl.kernel,
        out_type=jax.ShapeDtypeStruct((NC * NS, NPAD), jnp.float32),
        mesh=_mesh(),
        scratch_types=[
            pltpu.VMEM((EPW,), jnp.int32),
            pltpu.VMEM((NPAD,), jnp.float32),
        ],
        compiler_params=pltpu.CompilerParams(needs_layout_passes=False),
    )(_deg_body)


def _make_scatter(edge_split):
    DH = 128
    chunks = CHUNKS // 2 if edge_split else CHUNKS

    def body(table_hbm, src_hbm, dst_hbm, out_hbm,
             src_a, src_b, dst_v, rows_a, rows_b,
             sem_ga, sem_gb, sem_ia, sem_ib, sem_sa, sem_sb, acc):
        c = lax.axis_index("c")
        s = lax.axis_index("s")
        w = c * NS + s if edge_split else s
        pltpu.sync_copy(dst_hbm.at[w], dst_v)

        zeros = jnp.zeros((L,), jnp.float32)

        @pl.loop(0, CHUNK)
        def _zr(r):
            @pl.loop(0, DH // L)
            def _zc(k):
                rows_a[r, pl.ds(k * L, L)] = zeros

        for k in range(RPT // CHUNK):
            pltpu.sync_copy(rows_a, acc.at[pl.ds(s * RPT + k * CHUNK, CHUNK)])

        table = table_hbm if edge_split else table_hbm.at[c]
        last = chunks - 1

        def gather(sbuf, rbuf, sem):
            pltpu.async_copy(table.at[sbuf], rbuf, sem)

        def gwait(rbuf, sem):
            pltpu.make_async_copy(table.at[src_a], rbuf, sem).wait()

        def iload(jj, sbuf, sem):
            pltpu.async_copy(src_hbm.at[w, jj], sbuf, sem)

        def iwait(sbuf, sem):
            pltpu.make_async_copy(src_hbm.at[w, 0], sbuf, sem).wait()

        def scat(rbuf, jj, sem):
            pltpu.async_copy(rbuf, acc.at[dst_v.at[jj]], sem, add=True)

        def swait(rbuf, sem):
            pltpu.make_async_copy(table.at[src_a], rbuf, sem).wait()

        pltpu.sync_copy(src_hbm.at[w, 0], src_a)
        gather(src_a, rows_a, sem_ga)
        iload(1, src_b, sem_ib)
        iwait(src_b, sem_ib)
        plsc.subcore_barrier()

        gwait(rows_a, sem_ga)
        scat(rows_a, 0, sem_sa)
        gather(src_b, rows_b, sem_gb)
        iload(2, src_a, sem_ia)
        gwait(rows_b, sem_gb)
        scat(rows_b, 1, sem_sb)
        swait(rows_a, sem_sa)
        iwait(src_a, sem_ia)
        gather(src_a, rows_a, sem_ga)
        iload(3, src_b, sem_ib)
        iwait(src_b, sem_ib)

        @pl.loop(2, chunks, step=2)
        def _edge(j):
            gwait(rows_a, sem_ga)
            scat(rows_a, j, sem_sa)
            swait(rows_b, sem_sb)
            gather(src_b, rows_b, sem_gb)
            iload(jnp.minimum(j + 2, last), src_a, sem_ia)
            gwait(rows_b, sem_gb)
            scat(rows_b, j + 1, sem_sb)
            swait(rows_a, sem_sa)
            iwait(src_a, sem_ia)
            gather(src_a, rows_a, sem_ga)
            iload(jnp.minimum(j + 3, last), src_b, sem_ib)
            iwait(src_b, sem_ib)

        swait(rows_b, sem_sb)
        gwait(rows_a, sem_ga)
        plsc.subcore_barrier()
        pltpu.sync_copy(acc.at[pl.ds(s * RPT, RPT)],
                        out_hbm.at[c, pl.ds(s * RPT, RPT)])

    return functools.partial(
        pl.kernel,
        out_type=jax.ShapeDtypeStruct((NC, NPAD, DH), jnp.float32),
        mesh=_mesh(),
        scratch_types=[
            pltpu.VMEM((CHUNK,), jnp.int32),
            pltpu.VMEM((CHUNK,), jnp.int32),
            pltpu.VMEM((chunks, CHUNK), jnp.int32),
            pltpu.VMEM((CHUNK, DH), jnp.float32),
            pltpu.VMEM((CHUNK, DH), jnp.float32),
            pltpu.SemaphoreType.DMA,
            pltpu.SemaphoreType.DMA,
            pltpu.SemaphoreType.DMA,
            pltpu.SemaphoreType.DMA,
            pltpu.SemaphoreType.DMA,
            pltpu.SemaphoreType.DMA,
            pltpu.VMEM_SHARED((NPAD, DH), jnp.float32),
        ],
        compiler_params=pltpu.CompilerParams(needs_layout_passes=False),
    )(body)


_scatter_kernel = functools.cache(_make_scatter)


_BM = 512


def _prescale_body(hist_ref, x_ref, dinv_ref, ycat_ref):
    i = pl.program_id(0)
    deg = jnp.sum(hist_ref[...], axis=0) + 1.0
    dinv = lax.rsqrt(jnp.maximum(deg, 1.0))
    y = x_ref[...] * dinv[:, None]
    row = i * _BM + lax.broadcasted_iota(jnp.int32, (_BM, 1), 0)
    y = jnp.where(row < N, y, 0.0)
    dinv_ref[...] = dinv
    ycat_ref[0] = y[:, : IN_DIM // 2]
    ycat_ref[1] = y[:, IN_DIM // 2:]


def _prescale(hist, x):
    return pl.pallas_call(
        _prescale_body,
        grid=(NPAD // _BM,),
        in_specs=[pl.BlockSpec((NC * NS, _BM), lambda i: (0, i)),
                  pl.BlockSpec((_BM, IN_DIM), lambda i: (i, 0))],
        out_specs=[pl.BlockSpec((_BM,), lambda i: (i,)),
                   pl.BlockSpec((NC, _BM, IN_DIM // 2), lambda i: (0, i, 0))],
        out_shape=[jax.ShapeDtypeStruct((NPAD,), jnp.float32),
                   jax.ShapeDtypeStruct((NC, NPAD, IN_DIM // 2), jnp.float32)],
    )(hist, x)


def _enc_body(sx_ref, yx_ref, dinv_ref, b1_ref, w1_ref, wcat_ref, y2_ref):
    i = pl.program_id(0)
    sx = jnp.concatenate([sx_ref[0], sx_ref[1]], axis=1)
    yx = jnp.concatenate([yx_ref[0], yx_ref[1]], axis=1)
    dinv = dinv_ref[...]
    pre = dinv[:, None] * (sx + yx)
    h = jnp.maximum(
        jnp.dot(pre, w1_ref[...], preferred_element_type=jnp.float32)
        + b1_ref[...][None, :], 0.0)
    row = i * _BM + lax.broadcasted_iota(jnp.int32, (_BM, 1), 0)
    h = jnp.where(row < N, h, 0.0)
    c = jnp.dot(h, wcat_ref[...], preferred_element_type=jnp.float32)
    y2_ref[...] = c * dinv[:, None]


def _enc(sxcat, yxcat, dinv, b1, w1, wcat):
    return pl.pallas_call(
        _enc_body,
        grid=(NPAD // _BM,),
        in_specs=[pl.BlockSpec((NC, _BM, IN_DIM // 2), lambda i: (0, i, 0)),
                  pl.BlockSpec((NC, _BM, IN_DIM // 2), lambda i: (0, i, 0)),
                  pl.BlockSpec((_BM,), lambda i: (i,)),
                  pl.BlockSpec((FEAT,), lambda i: (0,)),
                  pl.BlockSpec((IN_DIM, FEAT), lambda i: (0, 0)),
                  pl.BlockSpec((FEAT, 2 * LAT), lambda i: (0, 0))],
        out_specs=pl.BlockSpec((_BM, 2 * LAT), lambda i: (i, 0)),
        out_shape=jax.ShapeDtypeStruct((NPAD, 2 * LAT), jnp.float32),
    )(sxcat, yxcat, dinv, b1, w1, wcat)


def _zcomp_body(s2_ref, y2_ref, dinv_ref, bcat_ref, gn_ref, z_ref):
    s = s2_ref[0] + s2_ref[1]
    y2 = y2_ref[...]
    o = dinv_ref[...][:, None] * (s + y2) + bcat_ref[...][None, :]
    xu = o[:, :LAT]
    xs = o[:, LAT:]
    z = gn_ref[...] * jnp.exp(xs) + xu
    z_ref[...] = (z * 0.7071067811865476).astype(jnp.bfloat16)


def _zcomp(s2cat, y2cat, dinv, bcat, gn):
    return pl.pallas_call(
        _zcomp_body,
        grid=(NPAD // _BM,),
        in_specs=[pl.BlockSpec((NC, _BM, 2 * LAT), lambda i: (0, i, 0)),
                  pl.BlockSpec((_BM, 2 * LAT), lambda i: (i, 0)),
                  pl.BlockSpec((_BM,), lambda i: (i,)),
                  pl.BlockSpec((2 * LAT,), lambda i: (0,)),
                  pl.BlockSpec((_BM, LAT), lambda i: (i, 0))],
        out_specs=pl.BlockSpec((_BM, LAT), lambda i: (i, 0)),
        out_shape=jax.ShapeDtypeStruct((NPAD, LAT), jnp.bfloat16),
    )(s2cat, y2cat, dinv, bcat, gn)


_BD = 2048
_BDN = 2560


def _decode_body(zr_ref, zc_ref, o_ref):
    p = lax.dot_general(zr_ref[...], zc_ref[...],
                        (((1,), (1,)), ((), ())),
                        preferred_element_type=jnp.float32)
    o_ref[...] = 0.5 * jnp.tanh(p) + 0.5


def _decode(z):
    return pl.pallas_call(
        _decode_body,
        grid=(NPAD // _BD, NPAD // _BDN),
        in_specs=[pl.BlockSpec((_BD, LAT), lambda i, j: (i, 0)),
                  pl.BlockSpec((_BDN, LAT), lambda i, j: (j, 0))],
        out_specs=pl.BlockSpec((_BD, _BDN), lambda i, j: (i, j)),
        out_shape=jax.ShapeDtypeStruct((N, N), jnp.float32),
    )(z, z)


def kernel(x, edge_index, W1, b1, Wmu, bmu, Wsig, bsig, gnoise):
    ei = edge_index.astype(jnp.int32)
    pad = jnp.asarray(N + np.arange(EPAD - E, dtype=np.int32) % (NPAD - N))
    src = jnp.concatenate([ei[0], pad])
    dst = jnp.concatenate([ei[1], pad])
    src_sc = src.reshape(NS, CHUNKS, CHUNK)
    dst_sc = dst.reshape(NS, CHUNKS, CHUNK)
    src_es = src.reshape(NC * NS, CHUNKS // 2, CHUNK)
    dst_es = dst.reshape(NC * NS, CHUNKS // 2, CHUNK)
    dst_deg = dst.reshape(NC * NS, EPW)

    wcat = jnp.concatenate([Wmu, Wsig], axis=1)
    bcat = jnp.concatenate([bmu, bsig])

    hist = _deg_kernel()(dst_deg)
    dinv, yxcat = _prescale(hist, x)
    sxcat = _scatter_kernel(False)(yxcat, src_sc, dst_sc)
    y2 = _enc(sxcat, yxcat, dinv, b1, W1, wcat)
    s2cat = _scatter_kernel(True)(y2, src_es, dst_es)
    z = _zcomp(s2cat, y2, dinv, bcat, gnoise)
    return _decode(z)


# speedup vs baseline: 1.0091x; 1.0084x over previous
import functools

import jax
import jax.numpy as jnp
import numpy as np
from jax import lax
from jax.experimental import pallas as pl
from jax.experimental.pallas import tpu as pltpu
from jax.experimental.pallas import tpu_sc as plsc

N = 10000
NPAD = 10240
IN_DIM = 256
FEAT = 256
LAT = 64
E = 160000
EPAD = 163840

NC, NS, L = 2, 16, 16
CHUNK = 128
CHUNKS = EPAD // NS // CHUNK
RPT = NPAD // NS
EPW = E // (NC * NS)


def _mesh():
    return plsc.VectorSubcoreMesh(
        core_axis_name="c", subcore_axis_name="s",
        num_cores=NC, num_subcores=NS)


def _deg_body(dst_hbm, hist_hbm, dst_v, hist_v):
    c = lax.axis_index("c")
    s = lax.axis_index("s")
    w = s * NC + c
    dst_v[pl.ds((EPW // L) * L, L)] = jnp.zeros((L,), jnp.int32)
    pltpu.sync_copy(dst_hbm.at[pl.ds(w * EPW, EPW)], dst_v.at[pl.ds(0, EPW)])
    zeros = jnp.zeros((L,), jnp.float32)
    ones = jnp.ones((L,), jnp.float32)

    @pl.loop(0, NPAD // L)
    def _zero(i):
        hist_v[pl.ds(i * L, L)] = zeros

    @pl.loop(0, EPW // L)
    def _count(i):
        idx = dst_v[pl.ds(i * L, L)]
        plsc.addupdate_scatter(hist_v, [idx], ones)

    tail = EPW - (EPW // L) * L
    if tail:
        idx = dst_v[pl.ds((EPW // L) * L, L)]
        mask = lax.iota(jnp.int32, L) < tail
        plsc.addupdate_scatter(hist_v, [idx], ones, mask=mask)

    pltpu.sync_copy(hist_v, hist_hbm.at[w])


@functools.cache
def _deg_kernel():
    return functools.partial(
        pl.kernel,
        out_type=jax.ShapeDtypeStruct((NC * NS, NPAD), jnp.float32),
        mesh=_mesh(),
        scratch_types=[
            pltpu.VMEM(((EPW // L + 1) * L,), jnp.int32),
            pltpu.VMEM((NPAD,), jnp.float32),
        ],
        compiler_params=pltpu.CompilerParams(needs_layout_passes=False),
    )(_deg_body)


def _make_scatter(edge_split):
    DH = 128
    chunks = CHUNKS // 2 if edge_split else CHUNKS

    def body(table_hbm, src_hbm, dst_hbm, out_hbm,
             src_a, src_b, dst_v, rows_a, rows_b,
             sem_ga, sem_gb, sem_ia, sem_ib, sem_sa, sem_sb, acc):
        c = lax.axis_index("c")
        s = lax.axis_index("s")
        w = c * NS + s if edge_split else s
        pltpu.sync_copy(dst_hbm.at[w], dst_v)

        zeros = jnp.zeros((L,), jnp.float32)

        @pl.loop(0, CHUNK)
        def _zr(r):
            @pl.loop(0, DH // L)
            def _zc(k):
                rows_a[r, pl.ds(k * L, L)] = zeros

        for k in range(RPT // CHUNK):
            pltpu.sync_copy(rows_a, acc.at[pl.ds(s * RPT + k * CHUNK, CHUNK)])

        table = table_hbm if edge_split else table_hbm.at[c]
        last = chunks - 1

        def gather(sbuf, rbuf, sem):
            pltpu.async_copy(table.at[sbuf], rbuf, sem)

        def gwait(rbuf, sem):
            pltpu.make_async_copy(table.at[src_a], rbuf, sem).wait()

        def iload(jj, sbuf, sem):
            pltpu.async_copy(src_hbm.at[w, jj], sbuf, sem)

        def iwait(sbuf, sem):
            pltpu.make_async_copy(src_hbm.at[w, 0], sbuf, sem).wait()

        def scat(rbuf, jj, sem):
            pltpu.async_copy(rbuf, acc.at[dst_v.at[jj]], sem, add=True)

        def swait(rbuf, sem):
            pltpu.make_async_copy(table.at[src_a], rbuf, sem).wait()

        pltpu.sync_copy(src_hbm.at[w, 0], src_a)
        gather(src_a, rows_a, sem_ga)
        iload(1, src_b, sem_ib)
        iwait(src_b, sem_ib)
        plsc.subcore_barrier()

        gwait(rows_a, sem_ga)
        scat(rows_a, 0, sem_sa)
        gather(src_b, rows_b, sem_gb)
        iload(2, src_a, sem_ia)
        gwait(rows_b, sem_gb)
        scat(rows_b, 1, sem_sb)
        swait(rows_a, sem_sa)
        iwait(src_a, sem_ia)
        gather(src_a, rows_a, sem_ga)
        iload(3, src_b, sem_ib)
        iwait(src_b, sem_ib)

        @pl.loop(2, chunks, step=2)
        def _edge(j):
            gwait(rows_a, sem_ga)
            scat(rows_a, j, sem_sa)
            swait(rows_b, sem_sb)
            gather(src_b, rows_b, sem_gb)
            iload(jnp.minimum(j + 2, last), src_a, sem_ia)
            gwait(rows_b, sem_gb)
            scat(rows_b, j + 1, sem_sb)
            swait(rows_a, sem_sa)
            iwait(src_a, sem_ia)
            gather(src_a, rows_a, sem_ga)
            iload(jnp.minimum(j + 3, last), src_b, sem_ib)
            iwait(src_b, sem_ib)

        swait(rows_b, sem_sb)
        gwait(rows_a, sem_ga)
        plsc.subcore_barrier()
        pltpu.sync_copy(acc.at[pl.ds(s * RPT, RPT)],
                        out_hbm.at[c, pl.ds(s * RPT, RPT)])

    return functools.partial(
        pl.kernel,
        out_type=jax.ShapeDtypeStruct((NC, NPAD, DH), jnp.float32),
        mesh=_mesh(),
        scratch_types=[
            pltpu.VMEM((CHUNK,), jnp.int32),
            pltpu.VMEM((CHUNK,), jnp.int32),
            pltpu.VMEM((chunks, CHUNK), jnp.int32),
            pltpu.VMEM((CHUNK, DH), jnp.float32),
            pltpu.VMEM((CHUNK, DH), jnp.float32),
            pltpu.SemaphoreType.DMA,
            pltpu.SemaphoreType.DMA,
            pltpu.SemaphoreType.DMA,
            pltpu.SemaphoreType.DMA,
            pltpu.SemaphoreType.DMA,
            pltpu.SemaphoreType.DMA,
            pltpu.VMEM_SHARED((NPAD, DH), jnp.float32),
        ],
        compiler_params=pltpu.CompilerParams(needs_layout_passes=False),
    )(body)


_scatter_kernel = functools.cache(_make_scatter)


_BM = 512


def _prescale_body(hist_ref, x_ref, dinv_ref, ycat_ref):
    i = pl.program_id(0)
    deg = jnp.sum(hist_ref[...], axis=0) + 1.0
    dinv = lax.rsqrt(jnp.maximum(deg, 1.0))
    y = x_ref[...] * dinv[:, None]
    row = i * _BM + lax.broadcasted_iota(jnp.int32, (_BM, 1), 0)
    y = jnp.where(row < N, y, 0.0)
    dinv_ref[...] = dinv
    ycat_ref[0] = y[:, : IN_DIM // 2]
    ycat_ref[1] = y[:, IN_DIM // 2:]


def _prescale(hist, x):
    return pl.pallas_call(
        _prescale_body,
        grid=(NPAD // _BM,),
        in_specs=[pl.BlockSpec((NC * NS, _BM), lambda i: (0, i)),
                  pl.BlockSpec((_BM, IN_DIM), lambda i: (i, 0))],
        out_specs=[pl.BlockSpec((_BM,), lambda i: (i,)),
                   pl.BlockSpec((NC, _BM, IN_DIM // 2), lambda i: (0, i, 0))],
        out_shape=[jax.ShapeDtypeStruct((NPAD,), jnp.float32),
                   jax.ShapeDtypeStruct((NC, NPAD, IN_DIM // 2), jnp.float32)],
    )(hist, x)


def _enc_body(sx_ref, yx_ref, dinv_ref, b1_ref, w1_ref, wcat_ref, y2_ref):
    i = pl.program_id(0)
    sx = jnp.concatenate([sx_ref[0], sx_ref[1]], axis=1)
    yx = jnp.concatenate([yx_ref[0], yx_ref[1]], axis=1)
    dinv = dinv_ref[...]
    pre = dinv[:, None] * (sx + yx)
    h = jnp.maximum(
        jnp.dot(pre, w1_ref[...], preferred_element_type=jnp.float32)
        + b1_ref[...][None, :], 0.0)
    row = i * _BM + lax.broadcasted_iota(jnp.int32, (_BM, 1), 0)
    h = jnp.where(row < N, h, 0.0)
    c = jnp.dot(h, wcat_ref[...], preferred_element_type=jnp.float32)
    y2_ref[...] = c * dinv[:, None]


def _enc(sxcat, yxcat, dinv, b1, w1, wcat):
    return pl.pallas_call(
        _enc_body,
        grid=(NPAD // _BM,),
        in_specs=[pl.BlockSpec((NC, _BM, IN_DIM // 2), lambda i: (0, i, 0)),
                  pl.BlockSpec((NC, _BM, IN_DIM // 2), lambda i: (0, i, 0)),
                  pl.BlockSpec((_BM,), lambda i: (i,)),
                  pl.BlockSpec((FEAT,), lambda i: (0,)),
                  pl.BlockSpec((IN_DIM, FEAT), lambda i: (0, 0)),
                  pl.BlockSpec((FEAT, 2 * LAT), lambda i: (0, 0))],
        out_specs=pl.BlockSpec((_BM, 2 * LAT), lambda i: (i, 0)),
        out_shape=jax.ShapeDtypeStruct((NPAD, 2 * LAT), jnp.float32),
    )(sxcat, yxcat, dinv, b1, w1, wcat)


def _zcomp_body(s2_ref, y2_ref, dinv_ref, bcat_ref, gn_ref, z_ref):
    s = s2_ref[0] + s2_ref[1]
    y2 = y2_ref[...]
    o = dinv_ref[...][:, None] * (s + y2) + bcat_ref[...][None, :]
    xu = o[:, :LAT]
    xs = o[:, LAT:]
    z = gn_ref[...] * jnp.exp(xs) + xu
    z_ref[...] = (z * 0.7071067811865476).astype(jnp.bfloat16)


def _zcomp(s2cat, y2cat, dinv, bcat, gn):
    return pl.pallas_call(
        _zcomp_body,
        grid=(NPAD // _BM,),
        in_specs=[pl.BlockSpec((NC, _BM, 2 * LAT), lambda i: (0, i, 0)),
                  pl.BlockSpec((_BM, 2 * LAT), lambda i: (i, 0)),
                  pl.BlockSpec((_BM,), lambda i: (i,)),
                  pl.BlockSpec((2 * LAT,), lambda i: (0,)),
                  pl.BlockSpec((_BM, LAT), lambda i: (i, 0))],
        out_specs=pl.BlockSpec((_BM, LAT), lambda i: (i, 0)),
        out_shape=jax.ShapeDtypeStruct((NPAD, LAT), jnp.bfloat16),
    )(s2cat, y2cat, dinv, bcat, gn)


_BD = 2048
_BDN = 2560


def _decode_body(zr_ref, zc_ref, o_ref):
    p = lax.dot_general(zr_ref[...], zc_ref[...],
                        (((1,), (1,)), ((), ())),
                        preferred_element_type=jnp.float32)
    o_ref[...] = 0.5 * jnp.tanh(p) + 0.5


def _decode(z):
    return pl.pallas_call(
        _decode_body,
        grid=(NPAD // _BD, NPAD // _BDN),
        in_specs=[pl.BlockSpec((_BD, LAT), lambda i, j: (i, 0)),
                  pl.BlockSpec((_BDN, LAT), lambda i, j: (j, 0))],
        out_specs=pl.BlockSpec((_BD, _BDN), lambda i, j: (i, j)),
        out_shape=jax.ShapeDtypeStruct((N, N), jnp.float32),
    )(z, z)


def kernel(x, edge_index, W1, b1, Wmu, bmu, Wsig, bsig, gnoise):
    ei = edge_index.astype(jnp.int32)
    pad = jnp.asarray(N + np.arange(EPAD - E, dtype=np.int32) % (NPAD - N))
    src = jnp.concatenate([ei[0], pad])
    dst = jnp.concatenate([ei[1], pad])
    src_sc = src.reshape(NS, CHUNKS, CHUNK)
    dst_sc = dst.reshape(NS, CHUNKS, CHUNK)
    src_es = src.reshape(NC * NS, CHUNKS // 2, CHUNK)
    dst_es = dst.reshape(NC * NS, CHUNKS // 2, CHUNK)

    wcat = jnp.concatenate([Wmu, Wsig], axis=1)
    bcat = jnp.concatenate([bmu, bsig])

    hist = _deg_kernel()(ei[1])
    dinv, yxcat = _prescale(hist, x)
    sxcat = _scatter_kernel(False)(yxcat, src_sc, dst_sc)
    y2 = _enc(sxcat, yxcat, dinv, b1, W1, wcat)
    s2cat = _scatter_kernel(True)(y2, src_es, dst_es)
    z = _zcomp(s2cat, y2, dinv, bcat, gnoise)
    return _decode(z)


# speedup vs baseline: 1.0136x; 1.0044x over previous
import functools

import jax
import jax.numpy as jnp
import numpy as np
from jax import lax
from jax.experimental import pallas as pl
from jax.experimental.pallas import tpu as pltpu
from jax.experimental.pallas import tpu_sc as plsc

N = 10000
NPAD = 10240
IN_DIM = 256
FEAT = 256
LAT = 64
E = 160000
EPAD = 163840

NC, NS, L = 2, 16, 16
CHUNK = 128
CHUNKS = EPAD // NS // CHUNK
RPT = NPAD // NS
EPW = E // (NC * NS)


def _mesh():
    return plsc.VectorSubcoreMesh(
        core_axis_name="c", subcore_axis_name="s",
        num_cores=NC, num_subcores=NS)


def _deg_body(dst_hbm, hist_hbm, dst_v, hist_v):
    c = lax.axis_index("c")
    s = lax.axis_index("s")
    w = s * NC + c
    dst_v[pl.ds((EPW // L) * L, L)] = jnp.zeros((L,), jnp.int32)
    pltpu.sync_copy(dst_hbm.at[pl.ds(w * EPW, EPW)], dst_v.at[pl.ds(0, EPW)])
    zeros = jnp.zeros((L,), jnp.float32)
    ones = jnp.ones((L,), jnp.float32)

    @pl.loop(0, NPAD // L)
    def _zero(i):
        hist_v[pl.ds(i * L, L)] = zeros

    @pl.loop(0, EPW // L)
    def _count(i):
        idx = dst_v[pl.ds(i * L, L)]
        plsc.addupdate_scatter(hist_v, [idx], ones)

    tail = EPW - (EPW // L) * L
    if tail:
        idx = dst_v[pl.ds((EPW // L) * L, L)]
        mask = lax.iota(jnp.int32, L) < tail
        plsc.addupdate_scatter(hist_v, [idx], ones, mask=mask)

    pltpu.sync_copy(hist_v, hist_hbm.at[w])


@functools.cache
def _deg_kernel():
    return functools.partial(
        pl.kernel,
        out_type=jax.ShapeDtypeStruct((NC * NS, NPAD), jnp.float32),
        mesh=_mesh(),
        scratch_types=[
            pltpu.VMEM(((EPW // L + 1) * L,), jnp.int32),
            pltpu.VMEM((NPAD,), jnp.float32),
        ],
        compiler_params=pltpu.CompilerParams(needs_layout_passes=False),
    )(_deg_body)


def _make_scatter(edge_split):
    DH = 128
    chunks = CHUNKS // 2 if edge_split else CHUNKS

    def body(table_hbm, src_hbm, dst_hbm, out_hbm,
             src_a, src_b, dst_v, rows_a, rows_b,
             sem_ga, sem_gb, sem_ia, sem_ib, sem_sa, sem_sb, acc):
        c = lax.axis_index("c")
        s = lax.axis_index("s")
        w = c * NS + s if edge_split else s
        pltpu.sync_copy(dst_hbm.at[w], dst_v)

        zeros = jnp.zeros((L,), jnp.float32)

        @pl.loop(0, CHUNK)
        def _zr(r):
            @pl.loop(0, DH // L)
            def _zc(k):
                rows_a[r, pl.ds(k * L, L)] = zeros

        for k in range(RPT // CHUNK):
            pltpu.async_copy(rows_a, acc.at[pl.ds(s * RPT + k * CHUNK, CHUNK)],
                             sem_sa)

        table = table_hbm if edge_split else table_hbm.at[c]
        last = chunks - 1

        def gather(sbuf, rbuf, sem):
            pltpu.async_copy(table.at[sbuf], rbuf, sem)

        def gwait(rbuf, sem):
            pltpu.make_async_copy(table.at[src_a], rbuf, sem).wait()

        def iload(jj, sbuf, sem):
            pltpu.async_copy(src_hbm.at[w, jj], sbuf, sem)

        def iwait(sbuf, sem):
            pltpu.make_async_copy(src_hbm.at[w, 0], sbuf, sem).wait()

        def scat(rbuf, jj, sem):
            pltpu.async_copy(rbuf, acc.at[dst_v.at[jj]], sem, add=True)

        def swait(rbuf, sem):
            pltpu.make_async_copy(table.at[src_a], rbuf, sem).wait()

        pltpu.sync_copy(src_hbm.at[w, 0], src_a)
        iload(1, src_b, sem_ib)
        for k in range(RPT // CHUNK):
            swait(rows_a, sem_sa)
        gather(src_a, rows_a, sem_ga)
        iwait(src_b, sem_ib)
        plsc.subcore_barrier()

        gwait(rows_a, sem_ga)
        scat(rows_a, 0, sem_sa)
        gather(src_b, rows_b, sem_gb)
        iload(2, src_a, sem_ia)
        gwait(rows_b, sem_gb)
        scat(rows_b, 1, sem_sb)
        swait(rows_a, sem_sa)
        iwait(src_a, sem_ia)
        gather(src_a, rows_a, sem_ga)
        iload(3, src_b, sem_ib)
        iwait(src_b, sem_ib)

        @pl.loop(2, chunks, step=2)
        def _edge(j):
            gwait(rows_a, sem_ga)
            scat(rows_a, j, sem_sa)
            swait(rows_b, sem_sb)
            gather(src_b, rows_b, sem_gb)
            iload(jnp.minimum(j + 2, last), src_a, sem_ia)
            gwait(rows_b, sem_gb)
            scat(rows_b, j + 1, sem_sb)
            swait(rows_a, sem_sa)
            iwait(src_a, sem_ia)
            gather(src_a, rows_a, sem_ga)
            iload(jnp.minimum(j + 3, last), src_b, sem_ib)
            iwait(src_b, sem_ib)

        swait(rows_b, sem_sb)
        gwait(rows_a, sem_ga)
        plsc.subcore_barrier()
        pltpu.sync_copy(acc.at[pl.ds(s * RPT, RPT)],
                        out_hbm.at[c, pl.ds(s * RPT, RPT)])

    return functools.partial(
        pl.kernel,
        out_type=jax.ShapeDtypeStruct((NC, NPAD, DH), jnp.float32),
        mesh=_mesh(),
        scratch_types=[
            pltpu.VMEM((CHUNK,), jnp.int32),
            pltpu.VMEM((CHUNK,), jnp.int32),
            pltpu.VMEM((chunks, CHUNK), jnp.int32),
            pltpu.VMEM((CHUNK, DH), jnp.float32),
            pltpu.VMEM((CHUNK, DH), jnp.float32),
            pltpu.SemaphoreType.DMA,
            pltpu.SemaphoreType.DMA,
            pltpu.SemaphoreType.DMA,
            pltpu.SemaphoreType.DMA,
            pltpu.SemaphoreType.DMA,
            pltpu.SemaphoreType.DMA,
            pltpu.VMEM_SHARED((NPAD, DH), jnp.float32),
        ],
        compiler_params=pltpu.CompilerParams(needs_layout_passes=False),
    )(body)


_scatter_kernel = functools.cache(_make_scatter)


_BM = 512


def _prescale_body(hist_ref, x_ref, dinv_ref, ycat_ref):
    i = pl.program_id(0)
    deg = jnp.sum(hist_ref[...], axis=0) + 1.0
    dinv = lax.rsqrt(jnp.maximum(deg, 1.0))
    y = x_ref[...] * dinv[:, None]
    row = i * _BM + lax.broadcasted_iota(jnp.int32, (_BM, 1), 0)
    y = jnp.where(row < N, y, 0.0)
    dinv_ref[...] = dinv
    ycat_ref[0] = y[:, : IN_DIM // 2]
    ycat_ref[1] = y[:, IN_DIM // 2:]


def _prescale(hist, x):
    return pl.pallas_call(
        _prescale_body,
        grid=(NPAD // _BM,),
        in_specs=[pl.BlockSpec((NC * NS, _BM), lambda i: (0, i)),
                  pl.BlockSpec((_BM, IN_DIM), lambda i: (i, 0))],
        out_specs=[pl.BlockSpec((_BM,), lambda i: (i,)),
                   pl.BlockSpec((NC, _BM, IN_DIM // 2), lambda i: (0, i, 0))],
        out_shape=[jax.ShapeDtypeStruct((NPAD,), jnp.float32),
                   jax.ShapeDtypeStruct((NC, NPAD, IN_DIM // 2), jnp.float32)],
    )(hist, x)


def _enc_body(sx_ref, yx_ref, dinv_ref, b1_ref, w1_ref, wcat_ref, y2_ref):
    i = pl.program_id(0)
    sx = jnp.concatenate([sx_ref[0], sx_ref[1]], axis=1)
    yx = jnp.concatenate([yx_ref[0], yx_ref[1]], axis=1)
    dinv = dinv_ref[...]
    pre = dinv[:, None] * (sx + yx)
    h = jnp.maximum(
        jnp.dot(pre, w1_ref[...], preferred_element_type=jnp.float32)
        + b1_ref[...][None, :], 0.0)
    row = i * _BM + lax.broadcasted_iota(jnp.int32, (_BM, 1), 0)
    h = jnp.where(row < N, h, 0.0)
    c = jnp.dot(h, wcat_ref[...], preferred_element_type=jnp.float32)
    y2_ref[...] = c * dinv[:, None]


def _enc(sxcat, yxcat, dinv, b1, w1, wcat):
    return pl.pallas_call(
        _enc_body,
        grid=(NPAD // _BM,),
        in_specs=[pl.BlockSpec((NC, _BM, IN_DIM // 2), lambda i: (0, i, 0)),
                  pl.BlockSpec((NC, _BM, IN_DIM // 2), lambda i: (0, i, 0)),
                  pl.BlockSpec((_BM,), lambda i: (i,)),
                  pl.BlockSpec((FEAT,), lambda i: (0,)),
                  pl.BlockSpec((IN_DIM, FEAT), lambda i: (0, 0)),
                  pl.BlockSpec((FEAT, 2 * LAT), lambda i: (0, 0))],
        out_specs=pl.BlockSpec((_BM, 2 * LAT), lambda i: (i, 0)),
        out_shape=jax.ShapeDtypeStruct((NPAD, 2 * LAT), jnp.float32),
    )(sxcat, yxcat, dinv, b1, w1, wcat)


def _zcomp_body(s2_ref, y2_ref, dinv_ref, bcat_ref, gn_ref, z_ref):
    s = s2_ref[0] + s2_ref[1]
    y2 = y2_ref[...]
    o = dinv_ref[...][:, None] * (s + y2) + bcat_ref[...][None, :]
    xu = o[:, :LAT]
    xs = o[:, LAT:]
    z = gn_ref[...] * jnp.exp(xs) + xu
    z_ref[...] = (z * 0.7071067811865476).astype(jnp.bfloat16)


def _zcomp(s2cat, y2cat, dinv, bcat, gn):
    return pl.pallas_call(
        _zcomp_body,
        grid=(NPAD // _BM,),
        in_specs=[pl.BlockSpec((NC, _BM, 2 * LAT), lambda i: (0, i, 0)),
                  pl.BlockSpec((_BM, 2 * LAT), lambda i: (i, 0)),
                  pl.BlockSpec((_BM,), lambda i: (i,)),
                  pl.BlockSpec((2 * LAT,), lambda i: (0,)),
                  pl.BlockSpec((_BM, LAT), lambda i: (i, 0))],
        out_specs=pl.BlockSpec((_BM, LAT), lambda i: (i, 0)),
        out_shape=jax.ShapeDtypeStruct((NPAD, LAT), jnp.bfloat16),
    )(s2cat, y2cat, dinv, bcat, gn)


_BD = 2048
_BDN = 2560


def _decode_body(zr_ref, zc_ref, o_ref):
    p = lax.dot_general(zr_ref[...], zc_ref[...],
                        (((1,), (1,)), ((), ())),
                        preferred_element_type=jnp.float32)
    o_ref[...] = 0.5 * jnp.tanh(p) + 0.5


def _decode(z):
    return pl.pallas_call(
        _decode_body,
        grid=(NPAD // _BD, NPAD // _BDN),
        in_specs=[pl.BlockSpec((_BD, LAT), lambda i, j: (i, 0)),
                  pl.BlockSpec((_BDN, LAT), lambda i, j: (j, 0))],
        out_specs=pl.BlockSpec((_BD, _BDN), lambda i, j: (i, j)),
        out_shape=jax.ShapeDtypeStruct((N, N), jnp.float32),
    )(z, z)


def kernel(x, edge_index, W1, b1, Wmu, bmu, Wsig, bsig, gnoise):
    ei = edge_index.astype(jnp.int32)
    pad = jnp.asarray(N + np.arange(EPAD - E, dtype=np.int32) % (NPAD - N))
    src = jnp.concatenate([ei[0], pad])
    dst = jnp.concatenate([ei[1], pad])
    src_sc = src.reshape(NS, CHUNKS, CHUNK)
    dst_sc = dst.reshape(NS, CHUNKS, CHUNK)
    src_es = src.reshape(NC * NS, CHUNKS // 2, CHUNK)
    dst_es = dst.reshape(NC * NS, CHUNKS // 2, CHUNK)

    wcat = jnp.concatenate([Wmu, Wsig], axis=1)
    bcat = jnp.concatenate([bmu, bsig])

    hist = _deg_kernel()(ei[1])
    dinv, yxcat = _prescale(hist, x)
    sxcat = _scatter_kernel(False)(yxcat, src_sc, dst_sc)
    y2 = _enc(sxcat, yxcat, dinv, b1, W1, wcat)
    s2cat = _scatter_kernel(True)(y2, src_es, dst_es)
    z = _zcomp(s2cat, y2, dinv, bcat, gnoise)
    return _decode(z)
